# Initial kernel scaffold; baseline (speedup 1.0000x reference)
import jax
import jax.numpy as jnp
from jax.experimental import pallas as pl


def kernel(sparse_indices, dense_inputs, embed_tables, dense_weights):
    raise NotImplementedError("write your pallas kernel here")


# speedup vs baseline: 1.2721x; 1.2721x over previous
import jax
import jax.numpy as jnp
from jax import lax
from jax.experimental import pallas as pl
from jax.experimental.pallas import tpu as pltpu
from jax.experimental.pallas import tpu_sc as plsc

BATCH = 16384
NF = 26
ND = 13
VOCAB = 100000
NC, NS, L = 2, 16, 16
NW = NC * NS
BPW = BATCH // NW
SPW = BPW * NF
DPW = BPW * ND


def _body(idx_hbm, dense_hbm, table_hbm, w_hbm, out_hbm,
          idx_v, vals_v, dense_v, w_v, out_v, sem_g, sem_i, sem_d):
    wid = lax.axis_index("s") * NC + lax.axis_index("c")
    base = wid * BPW

    cp_i = pltpu.async_copy(idx_hbm.at[pl.ds(base * NF, SPW)], idx_v, sem_i)
    cp_d = pltpu.async_copy(dense_hbm.at[pl.ds(base * ND, DPW)], dense_v, sem_d)
    cp_i.wait()

    iota = lax.iota(jnp.int32, L)

    def fld(i, carry):
        p0 = i * L
        f = jnp.remainder(p0 + iota, NF)
        idx_v[pl.ds(p0, L)] = idx_v[pl.ds(p0, L)] + f * VOCAB
        return carry
    lax.fori_loop(0, SPW // L, fld, 0)

    gcp = pltpu.async_copy(table_hbm.at[idx_v], vals_v, sem_g)

    cp_d.wait()
    pltpu.async_copy(w_hbm, w_v, sem_d).wait()
    wvec = w_v[:]

    def dense_chunk(c, carry):
        qvec = (c * L + iota) * ND
        acc = jnp.zeros((L,), jnp.float32)
        for d in range(ND):
            acc = acc + plsc.load_gather(dense_v, [qvec + d]) * wvec[d]
        out_v[pl.ds(c * L, L)] = acc
        return carry
    lax.fori_loop(0, BPW // L, dense_chunk, 0)

    gcp.wait()

    def emb_chunk(c, carry):
        pvec = (c * L + iota) * NF
        acc = out_v[pl.ds(c * L, L)]
        for f in range(NF):
            acc = acc + plsc.load_gather(vals_v, [pvec + f])
        out_v[pl.ds(c * L, L)] = acc
        return carry
    lax.fori_loop(0, BPW // L, emb_chunk, 0)

    pltpu.async_copy(out_v, out_hbm.at[pl.ds(base, BPW)], sem_g).wait()


_sc_call = pl.kernel(
    _body,
    out_type=jax.ShapeDtypeStruct((BATCH,), jnp.float32),
    mesh=plsc.VectorSubcoreMesh(core_axis_name="c", subcore_axis_name="s"),
    compiler_params=pltpu.CompilerParams(needs_layout_passes=False),
    scratch_types=[
        pltpu.VMEM((SPW,), jnp.int32),
        pltpu.VMEM((SPW,), jnp.float32),
        pltpu.VMEM((DPW,), jnp.float32),
        pltpu.VMEM((L,), jnp.float32),
        pltpu.VMEM((BPW,), jnp.float32),
        pltpu.SemaphoreType.DMA,
        pltpu.SemaphoreType.DMA,
        pltpu.SemaphoreType.DMA,
    ],
)


def kernel(sparse_indices, dense_inputs, embed_tables, dense_weights):
    idx_flat = sparse_indices.reshape(-1)
    dense_flat = dense_inputs.reshape(-1)
    table_flat = embed_tables.reshape(-1)
    w = jnp.pad(dense_weights.reshape(-1), (0, L - ND))
    out = _sc_call(idx_flat, dense_flat, table_flat, w)
    return out.reshape(BATCH, 1)


# speedup vs baseline: 3.2152x; 2.5275x over previous
import jax
import jax.numpy as jnp
from jax import lax
from jax.experimental import pallas as pl
from jax.experimental.pallas import tpu as pltpu
from jax.experimental.pallas import tpu_sc as plsc

BATCH = 16384
NF = 26
ND = 13
VOCAB = 100000
NC, NS, L = 2, 16, 16
NW = NC * NS
BPW = BATCH // NW
CPW = BPW // L

GROUPS = (tuple(range(0, 13)) + tuple(range(20, 26)), tuple(range(13, 20)))


def _make_body(fields, with_dense):
    nf = len(fields)

    def _body(*args):
        a = iter(args)
        idx_t = next(a)
        dense_t = next(a) if with_dense else None
        tables = [next(a) for _ in range(nf)]
        w_hbm = next(a) if with_dense else None
        partial = None if with_dense else next(a)
        out_hbm = next(a)
        idx_v = next(a)
        fidx_v = next(a)
        vals_v = next(a)
        dense_v = next(a) if with_dense else None
        w_v = next(a) if with_dense else None
        part_v = None if with_dense else next(a)
        out_v = next(a)
        sem_g = next(a)
        sem_i = next(a)
        sem_d = next(a)

        wid = lax.axis_index("s") * NC + lax.axis_index("c")
        base = wid * BPW

        cp_i = pltpu.async_copy(idx_t.at[:, pl.ds(base, BPW)], idx_v, sem_i)
        if with_dense:
            cp_d = pltpu.async_copy(dense_t.at[:, pl.ds(base, BPW)], dense_v, sem_d)
            cp_w = pltpu.async_copy(w_hbm, w_v, sem_d)
        else:
            cp_p = pltpu.async_copy(partial.at[pl.ds(base, BPW)], part_v, sem_d)
        cp_i.wait()

        iota = lax.iota(jnp.int32, L)

        gathers = []
        for k, f in enumerate(fields):
            frow = jnp.full((L,), f, jnp.int32)

            def fld(c, carry, frow=frow, k=k):
                cols = c * L + iota
                fidx_v[pl.ds(k * BPW + c * L, L)] = plsc.load_gather(idx_v, [frow, cols])
                return carry
            lax.fori_loop(0, CPW, fld, 0)
            gathers.append(pltpu.async_copy(
                tables[k].at[fidx_v.at[pl.ds(k * BPW, BPW)]],
                vals_v.at[pl.ds(k * BPW, BPW)], sem_g))

        if with_dense:
            cp_d.wait()
            cp_w.wait()
            wvec = w_v[:]

            def dense_chunk(c, carry):
                j0 = c * L
                cols = j0 + iota
                acc = jnp.zeros((L,), jnp.float32)
                for d in range(ND):
                    acc = acc + plsc.load_gather(
                        dense_v, [jnp.full((L,), d, jnp.int32), cols]) * wvec[d]
                out_v[pl.ds(j0, L)] = acc
                return carry
            lax.fori_loop(0, CPW, dense_chunk, 0)
        else:
            cp_p.wait()

            def seed_chunk(c, carry):
                j0 = c * L
                out_v[pl.ds(j0, L)] = part_v[pl.ds(j0, L)]
                return carry
            lax.fori_loop(0, CPW, seed_chunk, 0)

        for g in gathers:
            g.wait()

        def emb_chunk(c, carry):
            j0 = c * L
            acc = out_v[pl.ds(j0, L)]
            for k in range(nf):
                acc = acc + vals_v[pl.ds(k * BPW + j0, L)]
            out_v[pl.ds(j0, L)] = acc
            return carry
        lax.fori_loop(0, CPW, emb_chunk, 0)

        pltpu.async_copy(out_v, out_hbm.at[pl.ds(base, BPW)], sem_g).wait()

    return _body


def _make_call(fields, with_dense):
    nf = len(fields)
    scratch = [pltpu.VMEM((NF, BPW), jnp.int32),
               pltpu.VMEM((nf * BPW,), jnp.int32),
               pltpu.VMEM((nf * BPW,), jnp.float32)]
    if with_dense:
        scratch += [pltpu.VMEM((ND, BPW), jnp.float32),
                    pltpu.VMEM((L,), jnp.float32)]
    else:
        scratch += [pltpu.VMEM((BPW,), jnp.float32)]
    scratch += [pltpu.VMEM((BPW,), jnp.float32),
                pltpu.SemaphoreType.DMA,
                pltpu.SemaphoreType.DMA,
                pltpu.SemaphoreType.DMA]
    return pl.kernel(
        _make_body(fields, with_dense),
        out_type=jax.ShapeDtypeStruct((BATCH,), jnp.float32),
        mesh=plsc.VectorSubcoreMesh(core_axis_name="c", subcore_axis_name="s"),
        compiler_params=pltpu.CompilerParams(needs_layout_passes=False),
        scratch_types=scratch,
    )


_sc_calls = [_make_call(g, i == 0) for i, g in enumerate(GROUPS)]


def kernel(sparse_indices, dense_inputs, embed_tables, dense_weights):
    idx_t = sparse_indices.T
    dense_t = dense_inputs.T
    gtables = [
        jax.lax.optimization_barrier(tuple(embed_tables[f, :, 0] for f in g))
        for g in GROUPS
    ]
    w = jnp.pad(dense_weights.reshape(-1), (0, L - ND))
    part = _sc_calls[0](idx_t, dense_t, *gtables[0], w)
    for i in range(1, len(GROUPS)):
        part = _sc_calls[i](idx_t, *gtables[i], part)
    return part.reshape(BATCH, 1)
